# Initial kernel scaffold; baseline (speedup 1.0000x reference)
import jax
import jax.numpy as jnp
from jax.experimental import pallas as pl


def kernel(X, time_table, feature_table, W, b):
    raise NotImplementedError("write your pallas kernel here")


# speedup vs baseline: 6.6693x; 6.6693x over previous
import jax
import jax.numpy as jnp
from jax.experimental import pallas as pl
from jax.experimental.pallas import tpu as pltpu

_B, _T, _NEOF, _EMB = 4, 2048, 128, 32
_BT = 256


def _pe_kernel(x_ref, tt_ref, ftT_ref, w_row_ref, w_col_ref, b_ref, o_ref):
    tproj = jnp.sum(tt_ref[:] * w_row_ref[:], axis=1, keepdims=True)
    fproj = jnp.sum(ftT_ref[:] * w_col_ref[:], axis=0, keepdims=True)
    add = tproj + fproj + b_ref[0]
    o_ref[:] = x_ref[:] + add[None, :, :]


def kernel(X, time_table, feature_table, W, b):
    ftT = feature_table.T
    w_row = W.reshape(1, _EMB)
    w_col = W.reshape(_EMB, 1)
    grid = (_T // _BT,)
    return pl.pallas_call(
        _pe_kernel,
        grid=grid,
        in_specs=[
            pl.BlockSpec((_B, _BT, _NEOF), lambda i: (0, i, 0)),
            pl.BlockSpec((_BT, _EMB), lambda i: (i, 0)),
            pl.BlockSpec((_EMB, _NEOF), lambda i: (0, 0)),
            pl.BlockSpec((1, _EMB), lambda i: (0, 0)),
            pl.BlockSpec((_EMB, 1), lambda i: (0, 0)),
            pl.BlockSpec(memory_space=pltpu.SMEM),
        ],
        out_specs=pl.BlockSpec((_B, _BT, _NEOF), lambda i: (0, i, 0)),
        out_shape=jax.ShapeDtypeStruct((_B, _T, _NEOF), X.dtype),
    )(X, time_table, ftT, w_row, w_col, b)
